# baseline (speedup 1.0000x reference)
import functools

import jax
import jax.numpy as jnp
from jax import lax
from jax.experimental import pallas as pl
from jax.experimental.pallas import tpu as pltpu
from jax.experimental.pallas import tpu_sc as plsc

_E = 64
_K = 2
_CAP = 256
_SCALE = 1.0
_TBLK = 256
_NW = 32
_LANES = 16


def _router_body(h_ref, gw_ref, d0_ref, d1_ref, w0_ref, w1_ref, cnt_ref,
                 carry_ref):
    E = gw_ref.shape[0]
    tblk = h_ref.shape[0]

    @pl.when(pl.program_id(0) == 0)
    def _():
        carry_ref[...] = jnp.zeros_like(carry_ref)

    logits = lax.dot_general(gw_ref[...], h_ref[...],
                             (((1,), (1,)), ((), ())),
                             preferred_element_type=jnp.float32)
    m = jnp.max(logits, axis=0, keepdims=True)
    ex = jnp.exp(logits - m)
    probs = ex / jnp.sum(ex, axis=0, keepdims=True)

    eio = lax.broadcasted_iota(jnp.int32, (E, tblk), 0)
    m1 = jnp.max(probs, axis=0, keepdims=True)
    idx1 = jnp.min(jnp.where(probs == m1, eio, E), axis=0, keepdims=True)
    sel1 = eio == idx1
    probs2 = jnp.where(sel1, -1.0, probs)
    m2 = jnp.max(probs2, axis=0, keepdims=True)
    idx2 = jnp.min(jnp.where(probs2 == m2, eio, E), axis=0, keepdims=True)
    sel2 = eio == idx2

    onehot = sel1.astype(jnp.float32) + sel2.astype(jnp.float32)
    io_r = lax.broadcasted_iota(jnp.int32, (tblk, tblk), 0)
    io_c = lax.broadcasted_iota(jnp.int32, (tblk, tblk), 1)
    upper = (io_r < io_c).astype(jnp.float32)
    pos = jnp.dot(onehot, upper,
                  preferred_element_type=jnp.float32) + carry_ref[...]

    slot1 = jnp.sum(jnp.where(sel1, pos, 0.0), axis=0, keepdims=True)
    slot2 = jnp.sum(jnp.where(sel2, pos, 0.0), axis=0, keepdims=True)
    s1 = (slot1 + 0.5).astype(jnp.int32)
    s2 = (slot2 + 0.5).astype(jnp.int32)
    dump = E * _CAP
    ok1 = s1 < _CAP
    ok2 = s2 < _CAP
    d0 = jnp.where(ok1, idx1 * _CAP + s1, dump)
    d1 = jnp.where(ok2, idx2 * _CAP + s2, dump)
    w0 = jnp.where(ok1, m1 * _SCALE, 0.0)
    w1 = jnp.where(ok2, m2 * _SCALE, 0.0)

    rows = d0_ref.shape[0]
    d0_ref[...] = jnp.broadcast_to(d0, (rows, tblk))
    d1_ref[...] = jnp.broadcast_to(d1, (rows, tblk))
    w0_ref[...] = jnp.broadcast_to(w0, (rows, tblk))
    w1_ref[...] = jnp.broadcast_to(w1, (rows, tblk))
    carry_new = carry_ref[...] + jnp.sum(onehot, axis=1, keepdims=True)
    carry_ref[...] = carry_new
    ey_r = lax.broadcasted_iota(jnp.int32, (E, E), 0)
    ey_c = lax.broadcasted_iota(jnp.int32, (E, E), 1)
    eye = (ey_r == ey_c).astype(jnp.float32)
    cnt_row = lax.dot_general(carry_new, eye, (((0,), (0,)), ((), ())),
                              preferred_element_type=jnp.float32)
    cnt_i = (cnt_row + 0.5).astype(jnp.int32)
    n8 = jnp.minimum((jnp.minimum(cnt_i, _CAP) + 7) // 8 * 8, _CAP)
    lanes = cnt_ref.shape[1]
    n8w = jnp.concatenate(
        [n8, jnp.zeros((1, lanes - E), jnp.int32)], axis=1)
    lio = lax.broadcasted_iota(jnp.int32, (1, lanes), 1)
    nrows = jnp.where(lio == E, 8, n8w)
    cnt_ref[...] = jnp.broadcast_to(nrows, (cnt_ref.shape[0], lanes))


def _router(h, gate_w):
    T, H = h.shape
    E = gate_w.shape[0]
    nblk = T // _TBLK
    out_shape = [jax.ShapeDtypeStruct((8, T), jnp.int32),
                 jax.ShapeDtypeStruct((8, T), jnp.int32),
                 jax.ShapeDtypeStruct((8, T), jnp.float32),
                 jax.ShapeDtypeStruct((8, T), jnp.float32),
                 jax.ShapeDtypeStruct((8, 128), jnp.int32)]
    out_spec = pl.BlockSpec((8, _TBLK), lambda b: (0, b))
    cnt_spec = pl.BlockSpec((8, 128), lambda b: (0, 0))
    return pl.pallas_call(
        _router_body,
        grid=(nblk,),
        in_specs=[pl.BlockSpec((_TBLK, H), lambda b: (b, 0)),
                  pl.BlockSpec((E, H), lambda b: (0, 0))],
        out_specs=[out_spec, out_spec, out_spec, out_spec, cnt_spec],
        out_shape=out_shape,
        scratch_shapes=[pltpu.VMEM((E, 1), jnp.float32)],
        compiler_params=pltpu.CompilerParams(
            dimension_semantics=("arbitrary",)),
    )(h, gate_w)


def _mlp_body(num_experts, nrows_ref, x_any, gup_ref, dwn_ref, y_any,
              xloc, yloc, sx, sy):
    e = pl.program_id(0)
    I = dwn_ref.shape[1]
    cap = yloc.shape[0]

    def ranged_dma(loc, hbm, step, buf, sem, start, to_hbm):
        n8 = nrows_ref[0, step]
        for size in (256, 128, 64, 32, 16, 8):
            ofs = (n8 // (2 * size)) * (2 * size)

            @pl.when((n8 & size) != 0)
            def _(size=size, ofs=ofs):
                if buf is None:
                    vref = loc.at[pl.ds(ofs, size)]
                else:
                    vref = loc.at[buf, pl.ds(ofs, size)]
                href = hbm.at[pl.ds(step * cap + ofs, size)]
                cp = (pltpu.make_async_copy(vref, href, sem) if to_hbm
                      else pltpu.make_async_copy(href, vref, sem))
                if start:
                    cp.start()
                else:
                    cp.wait()

    @pl.when(e == 0)
    def _():
        ranged_dma(xloc, x_any, 0, 0, sx, start=True, to_hbm=False)

    ranged_dma(xloc, x_any, e, lax.rem(e, 2), sx, start=False, to_hbm=False)

    @pl.when(e < num_experts)
    def _():
        ranged_dma(xloc, x_any, e + 1, lax.rem(e + 1, 2), sx,
                   start=True, to_hbm=False)

    @pl.when(e > 0)
    def _():
        ranged_dma(yloc, y_any, e - 1, None, sy, start=False, to_hbm=True)

    xb = xloc[lax.rem(e, 2)].astype(jnp.bfloat16)
    gu = jnp.dot(xb, gup_ref[0].astype(jnp.bfloat16),
                 preferred_element_type=jnp.float32)
    gate = gu[:, :I]
    up = gu[:, I:]
    inter = (gate * lax.logistic(gate) * up).astype(jnp.bfloat16)
    y = jnp.dot(inter, dwn_ref[0].astype(jnp.bfloat16),
                preferred_element_type=jnp.float32)
    yloc[...] = jnp.where(e < num_experts, y, 0.0)

    ranged_dma(yloc, y_any, e, None, sy, start=True, to_hbm=True)

    @pl.when(e == num_experts)
    def _():
        ranged_dma(yloc, y_any, e, None, sy, start=False, to_hbm=True)


def _expert_mlp(nrows, xbuf, gate_up_proj, down_proj):
    E, H, I2 = gate_up_proj.shape
    I = I2 // 2
    rows = xbuf.shape[0]
    return pl.pallas_call(
        functools.partial(_mlp_body, E),
        grid_spec=pltpu.PrefetchScalarGridSpec(
            num_scalar_prefetch=1,
            grid=(E + 1,),
            in_specs=[pl.BlockSpec(memory_space=pl.ANY),
                      pl.BlockSpec((1, H, I2),
                                   lambda e, nr: (jnp.minimum(e, E - 1), 0, 0)),
                      pl.BlockSpec((1, I, H),
                                   lambda e, nr: (jnp.minimum(e, E - 1), 0, 0))],
            out_specs=pl.BlockSpec(memory_space=pl.ANY),
            scratch_shapes=[pltpu.VMEM((2, _CAP, H), jnp.float32),
                            pltpu.VMEM((_CAP, H), jnp.float32),
                            pltpu.SemaphoreType.DMA,
                            pltpu.SemaphoreType.DMA],
        ),
        out_shape=jax.ShapeDtypeStruct((rows, H), jnp.float32),
        compiler_params=pltpu.CompilerParams(
            dimension_semantics=("arbitrary",)),
    )(nrows, xbuf, gate_up_proj, down_proj)


def _make_dispatch(T, H, rows):
    tpw = T // _NW
    half = tpw // 2
    mesh = plsc.VectorSubcoreMesh(core_axis_name="c", subcore_axis_name="s")

    @functools.partial(
        pl.kernel, mesh=mesh,
        out_type=jax.ShapeDtypeStruct((rows, H), jnp.float32),
        scratch_types=[pltpu.VMEM((tpw, H), jnp.float32),
                       pltpu.VMEM((half,), jnp.int32),
                       pltpu.VMEM((half,), jnp.int32),
                       pltpu.VMEM((half,), jnp.int32),
                       pltpu.VMEM((half,), jnp.int32),
                       pltpu.SemaphoreType.DMA,
                       pltpu.SemaphoreType.DMA,
                       pltpu.SemaphoreType.DMA,
                       pltpu.SemaphoreType.DMA,
                       pltpu.SemaphoreType.DMA,
                       pltpu.SemaphoreType.DMA],
    )
    def dispatch(h_hbm, d0_hbm, d1_hbm, xbuf_hbm, hloc, d0a, d0b, d1a, d1b,
                 sh0, sh1, s0a, s0b, s1a, s1b):
        wid = lax.axis_index("s") * 2 + lax.axis_index("c")
        base = wid * tpw
        a0 = pltpu.async_copy(h_hbm.at[pl.ds(base, half)],
                              hloc.at[pl.ds(0, half)], sh0)
        a1 = pltpu.async_copy(h_hbm.at[pl.ds(base + half, half)],
                              hloc.at[pl.ds(half, half)], sh1)
        pltpu.sync_copy(d0_hbm.at[0, pl.ds(base, half)], d0a)
        pltpu.sync_copy(d0_hbm.at[0, pl.ds(base + half, half)], d0b)
        pltpu.sync_copy(d1_hbm.at[0, pl.ds(base, half)], d1a)
        pltpu.sync_copy(d1_hbm.at[0, pl.ds(base + half, half)], d1b)
        a0.wait()
        c0 = pltpu.async_copy(hloc.at[pl.ds(0, half)], xbuf_hbm.at[d0a], s0a)
        c1 = pltpu.async_copy(hloc.at[pl.ds(0, half)], xbuf_hbm.at[d1a], s1a)
        a1.wait()
        c2 = pltpu.async_copy(hloc.at[pl.ds(half, half)], xbuf_hbm.at[d0b],
                              s0b)
        c3 = pltpu.async_copy(hloc.at[pl.ds(half, half)], xbuf_hbm.at[d1b],
                              s1b)
        c0.wait()
        c1.wait()
        c2.wait()
        c3.wait()

    return dispatch


def _make_combine(T, H, dump):
    tpw = T // _NW
    chunk = _LANES
    nck = tpw // chunk
    unroll = 8
    mesh = plsc.VectorSubcoreMesh(core_axis_name="c", subcore_axis_name="s")

    @functools.partial(
        pl.kernel, mesh=mesh,
        out_type=jax.ShapeDtypeStruct((T, H), jnp.float32),
        scratch_types=[pltpu.VMEM((2, chunk, H), jnp.float32),
                       pltpu.VMEM((2, chunk, H), jnp.float32),
                       pltpu.VMEM((chunk, H), jnp.float32),
                       pltpu.VMEM((tpw,), jnp.int32),
                       pltpu.VMEM((tpw,), jnp.int32),
                       pltpu.VMEM((tpw,), jnp.float32),
                       pltpu.VMEM((tpw,), jnp.float32),
                       pltpu.SemaphoreType.DMA,
                       pltpu.SemaphoreType.DMA,
                       pltpu.SemaphoreType.DMA,
                       pltpu.SemaphoreType.DMA],
    )
    def combine(ybuf_hbm, d0_hbm, d1_hbm, w0_hbm, w1_hbm, out_hbm,
                y0loc, y1loc, oloc, d0v, d1v, w0v, w1v, s0a, s1a, s0b, s1b):
        wid = lax.axis_index("s") * 2 + lax.axis_index("c")
        base = wid * tpw
        sems = [(s0a, s1a), (s0b, s1b)]
        pltpu.sync_copy(d0_hbm.at[0, pl.ds(base, tpw)], d0v)
        pltpu.sync_copy(d1_hbm.at[0, pl.ds(base, tpw)], d1v)
        pltpu.sync_copy(w0_hbm.at[0, pl.ds(base, tpw)], w0v)
        pltpu.sync_copy(w1_hbm.at[0, pl.ds(base, tpw)], w1v)

        def start_gathers(c):
            sl = pl.ds(c * chunk, chunk)
            sg0, sg1 = sems[c % 2]
            g0 = pltpu.async_copy(ybuf_hbm.at[d0v.at[sl]],
                                  y0loc.at[c % 2], sg0)
            g1 = pltpu.async_copy(ybuf_hbm.at[d1v.at[sl]],
                                  y1loc.at[c % 2], sg1)
            return g0, g1

        pend = start_gathers(0)
        for c in range(nck):
            g0, g1 = pend
            if c + 1 < nck:
                nxt = start_gathers(c + 1)
            g0.wait()
            g1.wait()
            y0b = y0loc.at[c % 2]
            y1b = y1loc.at[c % 2]
            w0g = w0v[pl.ds(c * chunk, _LANES)]
            w1g = w1v[pl.ds(c * chunk, _LANES)]
            for i2 in range(_LANES):
                w0s = jnp.full((_LANES,), w0g[i2], jnp.float32)
                w1s = jnp.full((_LANES,), w1g[i2], jnp.float32)

                def j_body(j, carry, tok=i2, w0s=w0s, w1s=w1s,
                           y0b=y0b, y1b=y1b):
                    for u in range(unroll):
                        sl = pl.ds(j * (_LANES * unroll) + u * _LANES,
                                   _LANES)
                        oloc[tok, sl] = (y0b[tok, sl] * w0s
                                         + y1b[tok, sl] * w1s)
                    return carry

                lax.fori_loop(0, H // (_LANES * unroll), j_body, 0)

            pltpu.sync_copy(oloc, out_hbm.at[pl.ds(base + c * chunk, chunk)])
            if c + 1 < nck:
                pend = nxt

    return combine


def kernel(hidden_states, gate_w, gate_up_proj, down_proj):
    B, S, H = hidden_states.shape
    T = B * S
    E = gate_w.shape[0]
    rows = (E + 1) * _CAP

    h = hidden_states.reshape(T, H)

    d0, d1, w0, w1, nrows = _router(h, gate_w)
    xbuf = _make_dispatch(T, H, rows)(h, d0, d1)
    ybuf = _expert_mlp(nrows, xbuf, gate_up_proj, down_proj)
    out = _make_combine(T, H, E * _CAP)(ybuf, d0, d1, w0, w1)
    return out.reshape(B, S, H)


# speedup vs baseline: 1.0075x; 1.0075x over previous
import functools

import jax
import jax.numpy as jnp
from jax import lax
from jax.experimental import pallas as pl
from jax.experimental.pallas import tpu as pltpu
from jax.experimental.pallas import tpu_sc as plsc

_E = 64
_K = 2
_CAP = 256
_SCALE = 1.0
_TBLK = 256
_NW = 32
_LANES = 16


def _router_body(h_ref, gw_ref, d0_ref, d1_ref, w0_ref, w1_ref, cnt_ref,
                 carry_ref):
    E = gw_ref.shape[0]
    tblk = h_ref.shape[0]

    @pl.when(pl.program_id(0) == 0)
    def _():
        carry_ref[...] = jnp.zeros_like(carry_ref)

    logits = lax.dot_general(gw_ref[...], h_ref[...],
                             (((1,), (1,)), ((), ())),
                             preferred_element_type=jnp.float32)
    m = jnp.max(logits, axis=0, keepdims=True)
    ex = jnp.exp(logits - m)
    probs = ex / jnp.sum(ex, axis=0, keepdims=True)

    eio = lax.broadcasted_iota(jnp.int32, (E, tblk), 0)
    m1 = jnp.max(probs, axis=0, keepdims=True)
    idx1 = jnp.min(jnp.where(probs == m1, eio, E), axis=0, keepdims=True)
    sel1 = eio == idx1
    probs2 = jnp.where(sel1, -1.0, probs)
    m2 = jnp.max(probs2, axis=0, keepdims=True)
    idx2 = jnp.min(jnp.where(probs2 == m2, eio, E), axis=0, keepdims=True)
    sel2 = eio == idx2

    onehot = sel1.astype(jnp.float32) + sel2.astype(jnp.float32)
    io_r = lax.broadcasted_iota(jnp.int32, (tblk, tblk), 0)
    io_c = lax.broadcasted_iota(jnp.int32, (tblk, tblk), 1)
    upper = (io_r < io_c).astype(jnp.float32)
    pos = jnp.dot(onehot, upper,
                  preferred_element_type=jnp.float32) + carry_ref[...]

    slot1 = jnp.sum(jnp.where(sel1, pos, 0.0), axis=0, keepdims=True)
    slot2 = jnp.sum(jnp.where(sel2, pos, 0.0), axis=0, keepdims=True)
    s1 = (slot1 + 0.5).astype(jnp.int32)
    s2 = (slot2 + 0.5).astype(jnp.int32)
    dump = E * _CAP
    ok1 = s1 < _CAP
    ok2 = s2 < _CAP
    d0 = jnp.where(ok1, idx1 * _CAP + s1, dump)
    d1 = jnp.where(ok2, idx2 * _CAP + s2, dump)
    w0 = jnp.where(ok1, m1 * _SCALE, 0.0)
    w1 = jnp.where(ok2, m2 * _SCALE, 0.0)

    rows = d0_ref.shape[0]
    d0_ref[...] = jnp.broadcast_to(d0, (rows, tblk))
    d1_ref[...] = jnp.broadcast_to(d1, (rows, tblk))
    w0_ref[...] = jnp.broadcast_to(w0, (rows, tblk))
    w1_ref[...] = jnp.broadcast_to(w1, (rows, tblk))
    carry_new = carry_ref[...] + jnp.sum(onehot, axis=1, keepdims=True)
    carry_ref[...] = carry_new
    ey_r = lax.broadcasted_iota(jnp.int32, (E, E), 0)
    ey_c = lax.broadcasted_iota(jnp.int32, (E, E), 1)
    eye = (ey_r == ey_c).astype(jnp.float32)
    cnt_row = lax.dot_general(carry_new, eye, (((0,), (0,)), ((), ())),
                              preferred_element_type=jnp.float32)
    cnt_i = (cnt_row + 0.5).astype(jnp.int32)
    n8 = jnp.minimum((jnp.minimum(cnt_i, _CAP) + 7) // 8 * 8, _CAP)
    lanes = cnt_ref.shape[1]
    n8w = jnp.concatenate(
        [n8, jnp.zeros((1, lanes - E), jnp.int32)], axis=1)
    lio = lax.broadcasted_iota(jnp.int32, (1, lanes), 1)
    nrows = jnp.where(lio == E, 8, n8w)
    cnt_ref[...] = jnp.broadcast_to(nrows, (cnt_ref.shape[0], lanes))


def _router(h, gate_w):
    T, H = h.shape
    E = gate_w.shape[0]
    nblk = T // _TBLK
    out_shape = [jax.ShapeDtypeStruct((8, T), jnp.int32),
                 jax.ShapeDtypeStruct((8, T), jnp.int32),
                 jax.ShapeDtypeStruct((8, T), jnp.float32),
                 jax.ShapeDtypeStruct((8, T), jnp.float32),
                 jax.ShapeDtypeStruct((8, 128), jnp.int32)]
    out_spec = pl.BlockSpec((8, _TBLK), lambda b: (0, b))
    cnt_spec = pl.BlockSpec((8, 128), lambda b: (0, 0))
    return pl.pallas_call(
        _router_body,
        grid=(nblk,),
        in_specs=[pl.BlockSpec((_TBLK, H), lambda b: (b, 0)),
                  pl.BlockSpec((E, H), lambda b: (0, 0))],
        out_specs=[out_spec, out_spec, out_spec, out_spec, cnt_spec],
        out_shape=out_shape,
        scratch_shapes=[pltpu.VMEM((E, 1), jnp.float32)],
        compiler_params=pltpu.CompilerParams(
            dimension_semantics=("arbitrary",)),
    )(h, gate_w)


def _mlp_body(num_experts, nrows_ref, x_any, gup_ref, dwn_ref, y_any,
              xloc, yloc, sx, sy):
    e = pl.program_id(0)
    I = dwn_ref.shape[1]
    cap = yloc.shape[0]

    def ranged_dma(loc, hbm, step, buf, sem, start, to_hbm):
        n8 = nrows_ref[0, step]
        for size in (256, 128, 64, 32, 16, 8):
            ofs = (n8 // (2 * size)) * (2 * size)

            @pl.when((n8 & size) != 0)
            def _(size=size, ofs=ofs):
                if buf is None:
                    vref = loc.at[pl.ds(ofs, size)]
                else:
                    vref = loc.at[buf, pl.ds(ofs, size)]
                href = hbm.at[pl.ds(step * cap + ofs, size)]
                cp = (pltpu.make_async_copy(vref, href, sem) if to_hbm
                      else pltpu.make_async_copy(href, vref, sem))
                if start:
                    cp.start()
                else:
                    cp.wait()

    @pl.when(e == 0)
    def _():
        ranged_dma(xloc, x_any, 0, 0, sx, start=True, to_hbm=False)

    ranged_dma(xloc, x_any, e, lax.rem(e, 2), sx, start=False, to_hbm=False)

    @pl.when(e < num_experts)
    def _():
        ranged_dma(xloc, x_any, e + 1, lax.rem(e + 1, 2), sx,
                   start=True, to_hbm=False)

    @pl.when(e > 0)
    def _():
        ranged_dma(yloc, y_any, e - 1, None, sy, start=False, to_hbm=True)

    xb = xloc[lax.rem(e, 2)].astype(jnp.bfloat16)
    gu = jnp.dot(xb, gup_ref[0].astype(jnp.bfloat16),
                 preferred_element_type=jnp.float32)
    gate = gu[:, :I]
    up = gu[:, I:]
    inter = (gate * lax.logistic(gate) * up).astype(jnp.bfloat16)
    y = jnp.dot(inter, dwn_ref[0].astype(jnp.bfloat16),
                preferred_element_type=jnp.float32)
    yloc[...] = jnp.where(e < num_experts, y, 0.0)

    ranged_dma(yloc, y_any, e, None, sy, start=True, to_hbm=True)

    @pl.when(e == num_experts)
    def _():
        ranged_dma(yloc, y_any, e, None, sy, start=False, to_hbm=True)


def _expert_mlp(nrows, xbuf, gate_up_proj, down_proj):
    E, H, I2 = gate_up_proj.shape
    I = I2 // 2
    rows = xbuf.shape[0]
    return pl.pallas_call(
        functools.partial(_mlp_body, E),
        grid_spec=pltpu.PrefetchScalarGridSpec(
            num_scalar_prefetch=1,
            grid=(E + 1,),
            in_specs=[pl.BlockSpec(memory_space=pl.ANY),
                      pl.BlockSpec((1, H, I2),
                                   lambda e, nr: (jnp.minimum(e, E - 1), 0, 0)),
                      pl.BlockSpec((1, I, H),
                                   lambda e, nr: (jnp.minimum(e, E - 1), 0, 0))],
            out_specs=pl.BlockSpec(memory_space=pl.ANY),
            scratch_shapes=[pltpu.VMEM((2, _CAP, H), jnp.float32),
                            pltpu.VMEM((_CAP, H), jnp.float32),
                            pltpu.SemaphoreType.DMA,
                            pltpu.SemaphoreType.DMA],
        ),
        out_shape=jax.ShapeDtypeStruct((rows, H), jnp.float32),
        compiler_params=pltpu.CompilerParams(
            dimension_semantics=("arbitrary",)),
    )(nrows, xbuf, gate_up_proj, down_proj)


def _make_dispatch(T, H, rows):
    tpw = T // _NW
    mesh = plsc.VectorSubcoreMesh(core_axis_name="c", subcore_axis_name="s")

    @functools.partial(
        pl.kernel, mesh=mesh,
        out_type=jax.ShapeDtypeStruct((rows, H), jnp.float32),
        scratch_types=[pltpu.VMEM((tpw, H), jnp.float32),
                       pltpu.VMEM((tpw,), jnp.int32),
                       pltpu.VMEM((tpw,), jnp.int32),
                       pltpu.SemaphoreType.DMA,
                       pltpu.SemaphoreType.DMA],
    )
    def dispatch(h_hbm, d0_hbm, d1_hbm, xbuf_hbm, hloc, d0v, d1v, s0, s1):
        wid = lax.axis_index("s") * 2 + lax.axis_index("c")
        base = wid * tpw
        pltpu.sync_copy(h_hbm.at[pl.ds(base, tpw)], hloc)
        pltpu.sync_copy(d0_hbm.at[0, pl.ds(base, tpw)], d0v)
        pltpu.sync_copy(d1_hbm.at[0, pl.ds(base, tpw)], d1v)
        c0 = pltpu.async_copy(hloc, xbuf_hbm.at[d0v], s0)
        c1 = pltpu.async_copy(hloc, xbuf_hbm.at[d1v], s1)
        c0.wait()
        c1.wait()

    return dispatch


def _make_combine(T, H, dump):
    tpw = T // _NW
    chunk = _LANES
    nck = tpw // chunk
    unroll = 8
    mesh = plsc.VectorSubcoreMesh(core_axis_name="c", subcore_axis_name="s")

    @functools.partial(
        pl.kernel, mesh=mesh,
        out_type=jax.ShapeDtypeStruct((T, H), jnp.float32),
        scratch_types=[pltpu.VMEM((2, chunk, H), jnp.float32),
                       pltpu.VMEM((2, chunk, H), jnp.float32),
                       pltpu.VMEM((chunk, H), jnp.float32),
                       pltpu.VMEM((tpw,), jnp.int32),
                       pltpu.VMEM((tpw,), jnp.int32),
                       pltpu.VMEM((tpw,), jnp.float32),
                       pltpu.VMEM((tpw,), jnp.float32),
                       pltpu.SemaphoreType.DMA,
                       pltpu.SemaphoreType.DMA,
                       pltpu.SemaphoreType.DMA,
                       pltpu.SemaphoreType.DMA],
    )
    def combine(ybuf_hbm, d0_hbm, d1_hbm, w0_hbm, w1_hbm, out_hbm,
                y0loc, y1loc, oloc, d0v, d1v, w0v, w1v, s0a, s1a, s0b, s1b):
        wid = lax.axis_index("s") * 2 + lax.axis_index("c")
        base = wid * tpw
        sems = [(s0a, s1a), (s0b, s1b)]
        ad0 = pltpu.async_copy(d0_hbm.at[0, pl.ds(base, tpw)], d0v, s0a)
        ad1 = pltpu.async_copy(d1_hbm.at[0, pl.ds(base, tpw)], d1v, s1a)
        aw0 = pltpu.async_copy(w0_hbm.at[0, pl.ds(base, tpw)], w0v, s0b)
        aw1 = pltpu.async_copy(w1_hbm.at[0, pl.ds(base, tpw)], w1v, s1b)
        ad0.wait()
        ad1.wait()
        aw0.wait()
        aw1.wait()

        def start_gathers(c):
            sl = pl.ds(c * chunk, chunk)
            sg0, sg1 = sems[c % 2]
            g0 = pltpu.async_copy(ybuf_hbm.at[d0v.at[sl]],
                                  y0loc.at[c % 2], sg0)
            g1 = pltpu.async_copy(ybuf_hbm.at[d1v.at[sl]],
                                  y1loc.at[c % 2], sg1)
            return g0, g1

        pend = start_gathers(0)
        for c in range(nck):
            g0, g1 = pend
            if c + 1 < nck:
                nxt = start_gathers(c + 1)
            g0.wait()
            g1.wait()
            y0b = y0loc.at[c % 2]
            y1b = y1loc.at[c % 2]
            w0g = w0v[pl.ds(c * chunk, _LANES)]
            w1g = w1v[pl.ds(c * chunk, _LANES)]
            for i2 in range(_LANES):
                w0s = jnp.full((_LANES,), w0g[i2], jnp.float32)
                w1s = jnp.full((_LANES,), w1g[i2], jnp.float32)

                def j_body(j, carry, tok=i2, w0s=w0s, w1s=w1s,
                           y0b=y0b, y1b=y1b):
                    for u in range(unroll):
                        sl = pl.ds(j * (_LANES * unroll) + u * _LANES,
                                   _LANES)
                        oloc[tok, sl] = (y0b[tok, sl] * w0s
                                         + y1b[tok, sl] * w1s)
                    return carry

                lax.fori_loop(0, H // (_LANES * unroll), j_body, 0)

            pltpu.sync_copy(oloc, out_hbm.at[pl.ds(base + c * chunk, chunk)])
            if c + 1 < nck:
                pend = nxt

    return combine


def kernel(hidden_states, gate_w, gate_up_proj, down_proj):
    B, S, H = hidden_states.shape
    T = B * S
    E = gate_w.shape[0]
    rows = (E + 1) * _CAP

    h = hidden_states.reshape(T, H)

    d0, d1, w0, w1, nrows = _router(h, gate_w)
    xbuf = _make_dispatch(T, H, rows)(h, d0, d1)
    ybuf = _expert_mlp(nrows, xbuf, gate_up_proj, down_proj)
    out = _make_combine(T, H, E * _CAP)(ybuf, d0, d1, w0, w1)
    return out.reshape(B, S, H)
